# baseline (speedup 1.0000x reference)
import jax
import jax.numpy as jnp
from jax import lax
from jax.experimental import pallas as pl
from jax.experimental.pallas import tpu as pltpu
from jax.experimental.pallas import tpu_sc as plsc

TOPK = 16
LANES = 16
NEG_INF = float("-inf")
CH = 8
NBUF = 2
BPW = 2


def _xor_reduce(v, op):
    iota = lax.broadcasted_iota(jnp.int32, (LANES,), 0)
    for s in (8, 4, 2, 1):
        v = op(v, v.at[iota ^ s].get(mode="promise_in_bounds"))
    return v


def _sc_consensus_body(
    lite_hbm, in_hbm, out_hbm,
    lite_buf, rows_v, idx_v, out_v, scores_v,
    lsem0, lsem1, gsem0, gsem1,
):
    D = in_hbm.shape[1]
    T = 64
    nc = 2
    wid = lax.axis_index("s") * nc + lax.axis_index("c")
    row0 = wid * BPW * T
    nch = BPW * T // CH
    lsems = [lsem0, lsem1]
    iota = lax.broadcasted_iota(jnp.int32, (LANES,), 0)

    def lite_copy(cc, b):
        return pltpu.make_async_copy(
            lite_hbm.at[pl.ds(row0 + cc * CH, CH)], lite_buf.at[b], lsems[b]
        )

    for b in range(NBUF):
        lite_copy(b, b).start()

    @pl.loop(0, nch, step=NBUF)
    def _chunks(c):
        for b in range(NBUF):
            cc = c + b
            lite_copy(cc, b).wait()
            g = cc // 2
            lane0 = (cc % 2) * CH
            sv = scores_v[g, :]
            for r in range(CH):
                accs = [lite_buf[b, r, pl.ds(j * LANES, LANES)] for j in range(4)]
                for col in range(4, D // LANES):
                    accs[col % 4] = jnp.maximum(
                        accs[col % 4], lite_buf[b, r, pl.ds(col * LANES, LANES)]
                    )
                m = jnp.maximum(
                    jnp.maximum(accs[0], accs[1]), jnp.maximum(accs[2], accs[3])
                )
                rmax = _xor_reduce(m, jnp.maximum)
                sv = jnp.where(iota == lane0 + r, rmax, sv)
            scores_v[g, :] = sv

            @pl.when(cc + NBUF < nch)
            def _():
                lite_copy(cc + NBUF, b).start()

    gsems = [gsem0, gsem1]
    big = jnp.int32(2**30)
    for bb in range(BPW):
        svecs = [scores_v[bb * 4 + j, :] for j in range(4)]
        idx_acc = jnp.zeros((LANES,), jnp.int32)
        for k in range(TOPK):
            m = jnp.maximum(
                jnp.maximum(svecs[0], svecs[1]), jnp.maximum(svecs[2], svecs[3])
            )
            mx = _xor_reduce(m, jnp.maximum)
            cands = [
                jnp.where(svecs[j] == mx, iota + j * LANES, big) for j in range(4)
            ]
            cmin = jnp.minimum(
                jnp.minimum(cands[0], cands[1]), jnp.minimum(cands[2], cands[3])
            )
            t = _xor_reduce(cmin, jnp.minimum)
            idx_acc = jnp.where(iota == k, (wid * BPW + bb) * T + t, idx_acc)
            for j in range(4):
                svecs[j] = jnp.where(iota + j * LANES == t, NEG_INF, svecs[j])
        idx_v[bb, :] = idx_acc

    gathers = [
        pltpu.make_async_copy(in_hbm.at[idx_v.at[bb]], rows_v.at[bb], gsems[bb])
        for bb in range(BPW)
    ]
    for gth in gathers:
        gth.start()
    for bb in range(BPW):
        gathers[bb].wait()

        @pl.loop(0, D // LANES)
        def _mean(cidx):
            sl = pl.ds(cidx * LANES, LANES)
            acc = rows_v[bb, 0, sl]
            for r in range(1, TOPK):
                acc = acc + rows_v[bb, r, sl]
            out_v[bb, sl] = acc * (1.0 / TOPK)

    pltpu.sync_copy(out_v, out_hbm.at[pl.ds(wid * BPW, BPW)])


@jax.jit
def kernel(input, lite_input):
    B, T, D = input.shape
    lite_rows = lite_input.reshape(B * T, D)
    input_rows = input.reshape(B * T, D)

    sc_consensus = pl.kernel(
        _sc_consensus_body,
        out_type=jax.ShapeDtypeStruct((B, D), jnp.float32),
        mesh=plsc.VectorSubcoreMesh(core_axis_name="c", subcore_axis_name="s"),
        scratch_types=[
            pltpu.VMEM((NBUF, CH, D), jnp.float32),
            pltpu.VMEM((BPW, TOPK, D), jnp.float32),
            pltpu.VMEM((BPW, TOPK), jnp.int32),
            pltpu.VMEM((BPW, D), jnp.float32),
            pltpu.VMEM((BPW * 4, LANES), jnp.float32),
            pltpu.SemaphoreType.DMA,
            pltpu.SemaphoreType.DMA,
            pltpu.SemaphoreType.DMA,
            pltpu.SemaphoreType.DMA,
        ],
    )
    out = sc_consensus(lite_rows, input_rows)

    return out.reshape(B, 1, D)


# speedup vs baseline: 2.4175x; 2.4175x over previous
import jax
import jax.numpy as jnp
from jax import lax
from jax.experimental import pallas as pl
from jax.experimental.pallas import tpu as pltpu
from jax.experimental.pallas import tpu_sc as plsc

TOPK = 16
LANES = 16
NEG_INF = float("-inf")
BB = 8
BPW = 2
SPAD = 128


def _scores_body(lite_ref, scores_ref):
    s = jnp.max(lite_ref[...], axis=2)
    pad = jnp.zeros((BB, SPAD - s.shape[1]), jnp.float32)
    scores_ref[...] = jnp.concatenate([s, pad], axis=1)


def _xor_reduce(v, op):
    iota = lax.broadcasted_iota(jnp.int32, (LANES,), 0)
    for s in (8, 4, 2, 1):
        v = op(v, v.at[iota ^ s].get(mode="promise_in_bounds"))
    return v


def _sc_topk_gather_mean_body(
    scores_hbm, in_hbm, out_hbm, sc_v, rows_v, idx_v, out_v, gsem0, gsem1
):
    D = in_hbm.shape[1]
    T = 64
    nc = 2
    wid = lax.axis_index("s") * nc + lax.axis_index("c")
    iota = lax.broadcasted_iota(jnp.int32, (LANES,), 0)
    pltpu.sync_copy(scores_hbm.at[pl.ds(wid * BPW, BPW)], sc_v)

    gsems = [gsem0, gsem1]
    big = jnp.int32(2**30)
    for bb in range(BPW):
        svecs = [sc_v[bb, pl.ds(j * LANES, LANES)] for j in range(4)]
        idx_acc = jnp.zeros((LANES,), jnp.int32)
        for k in range(TOPK):
            m = jnp.maximum(
                jnp.maximum(svecs[0], svecs[1]), jnp.maximum(svecs[2], svecs[3])
            )
            mx = _xor_reduce(m, jnp.maximum)
            cands = [
                jnp.where(svecs[j] == mx, iota + j * LANES, big) for j in range(4)
            ]
            cmin = jnp.minimum(
                jnp.minimum(cands[0], cands[1]), jnp.minimum(cands[2], cands[3])
            )
            t = _xor_reduce(cmin, jnp.minimum)
            idx_acc = jnp.where(iota == k, (wid * BPW + bb) * T + t, idx_acc)
            for j in range(4):
                svecs[j] = jnp.where(iota + j * LANES == t, NEG_INF, svecs[j])
        idx_v[bb, :] = idx_acc

    gathers = [
        pltpu.make_async_copy(in_hbm.at[idx_v.at[bb]], rows_v.at[bb], gsems[bb])
        for bb in range(BPW)
    ]
    for gth in gathers:
        gth.start()
    for bb in range(BPW):
        gathers[bb].wait()

        @pl.loop(0, D // LANES)
        def _mean(cidx):
            sl = pl.ds(cidx * LANES, LANES)
            acc = rows_v[bb, 0, sl]
            for r in range(1, TOPK):
                acc = acc + rows_v[bb, r, sl]
            out_v[bb, sl] = acc * (1.0 / TOPK)

    pltpu.sync_copy(out_v, out_hbm.at[pl.ds(wid * BPW, BPW)])


@jax.jit
def kernel(input, lite_input):
    B, T, D = input.shape

    scores = pl.pallas_call(
        _scores_body,
        grid=(B // BB,),
        in_specs=[pl.BlockSpec((BB, T, D), lambda b: (b, 0, 0))],
        out_specs=pl.BlockSpec((BB, SPAD), lambda b: (b, 0)),
        out_shape=jax.ShapeDtypeStruct((B, SPAD), jnp.float32),
    )(lite_input)

    input_rows = input.reshape(B * T, D)

    sc_stage = pl.kernel(
        _sc_topk_gather_mean_body,
        out_type=jax.ShapeDtypeStruct((B, D), jnp.float32),
        mesh=plsc.VectorSubcoreMesh(core_axis_name="c", subcore_axis_name="s"),
        scratch_types=[
            pltpu.VMEM((BPW, SPAD), jnp.float32),
            pltpu.VMEM((BPW, TOPK, D), jnp.float32),
            pltpu.VMEM((BPW, TOPK), jnp.int32),
            pltpu.VMEM((BPW, D), jnp.float32),
            pltpu.SemaphoreType.DMA,
            pltpu.SemaphoreType.DMA,
        ],
    )
    out = sc_stage(scores, input_rows)

    return out.reshape(B, 1, D)
